# Initial kernel scaffold; baseline (speedup 1.0000x reference)
import jax
import jax.numpy as jnp
from jax.experimental import pallas as pl


def kernel(time, labels, label_emb_table, channels):
    raise NotImplementedError("write your pallas kernel here")


# speedup vs baseline: 1.4038x; 1.4038x over previous
import functools

import jax
import jax.numpy as jnp
from jax import lax
from jax.experimental import pallas as pl
from jax.experimental.pallas import tpu as pltpu
from jax.experimental.pallas import tpu_sc as plsc

B = 16384
C = 256
HALF = C // 2

_info = plsc.get_sparse_core_info()
_NC, _NS = _info.num_cores, _info.num_subcores
_NW = _NC * _NS
_B_PER_W = B // _NW
_CHUNK = 128
_N_CHUNKS = _B_PER_W // _CHUNK


def _make_sc_gather():
    mesh = plsc.VectorSubcoreMesh(core_axis_name="c", subcore_axis_name="s")

    @functools.partial(
        pl.kernel,
        mesh=mesh,
        out_type=jax.ShapeDtypeStruct((B, C), jnp.float32),
        scratch_types=[
            pltpu.VMEM((_CHUNK,), jnp.int32),
            pltpu.VMEM((_CHUNK, C), jnp.float32),
            pltpu.SemaphoreType.DMA,
        ],
    )
    def gather_k(idx_hbm, table_hbm, out_hbm, idx_v, rows_v, sem):
        wid = lax.axis_index("s") * _NC + lax.axis_index("c")
        for j in range(_N_CHUNKS):
            base = wid * _B_PER_W + j * _CHUNK
            pltpu.sync_copy(idx_hbm.at[pl.ds(base, _CHUNK)], idx_v)
            pltpu.async_copy(table_hbm.at[idx_v], rows_v, sem).wait()
            pltpu.sync_copy(rows_v, out_hbm.at[pl.ds(base, _CHUNK)])

    return gather_k


_sc_gather = _make_sc_gather()

_BLK = 2048


def _tc_body(time_ref, invf_ref, emb_ref, out_ref):
    t = time_ref[...]
    f = invf_ref[...]
    arg = t * f
    out_ref[:, :HALF] = jnp.sin(arg) + emb_ref[:, :HALF]
    out_ref[:, HALF:] = jnp.cos(arg) + emb_ref[:, HALF:]


def _tc_combine(time, inv_freq, emb):
    return pl.pallas_call(
        _tc_body,
        out_shape=jax.ShapeDtypeStruct((B, C), jnp.float32),
        grid=(B // _BLK,),
        in_specs=[
            pl.BlockSpec((_BLK, 1), lambda i: (i, 0)),
            pl.BlockSpec((1, HALF), lambda i: (0, 0)),
            pl.BlockSpec((_BLK, C), lambda i: (i, 0)),
        ],
        out_specs=pl.BlockSpec((_BLK, C), lambda i: (i, 0)),
    )(time, inv_freq, emb)


def kernel(time, labels, label_emb_table, channels):
    labels_i = labels.astype(jnp.int32)
    emb = _sc_gather(labels_i, label_emb_table)
    inv_freq = 1.0 / (
        10000.0
        ** (jnp.arange(0, C, 2, dtype=jnp.float32)
            / jnp.asarray(channels).astype(jnp.float32))
    )
    return _tc_combine(time, inv_freq.reshape(1, HALF), emb)
